# baseline (speedup 1.0000x reference)
import functools
import jax
import jax.numpy as jnp
from jax import lax
from jax.experimental import pallas as pl
from jax.experimental.pallas import tpu as pltpu
from jax.experimental.pallas import tpu_sc as plsc

B = 8
N = 131072
C = 16
K = 16384
NT = 4
CH = N // NT
SEL = K
CAP = SEL + 16
DUMP = 4 * SEL
QCH = K // NT

_BLKA = 16384


def _prep_body(scores_ref, ori_ref, keys_ref):
    s = scores_ref[...]
    m = jnp.max(s, axis=1)
    ori_ref[...] = jax.nn.sigmoid(m)[:, None, :]
    bu = lax.bitcast_convert_type(m, jnp.uint32)
    sign = bu >> 31
    key = jnp.where(sign == 1, bu, ~bu & jnp.uint32(0x7FFFFFFF))
    keys_ref[...] = lax.bitcast_convert_type(key, jnp.int32).reshape(_BLKA)


def _prep(scores):
    return pl.pallas_call(
        _prep_body,
        grid=(B, N // _BLKA),
        in_specs=[pl.BlockSpec((1, 3, _BLKA), lambda b, i: (b, 0, i))],
        out_specs=[
            pl.BlockSpec((1, 1, _BLKA), lambda b, i: (b, 0, i)),
            pl.BlockSpec((_BLKA,), lambda b, i: (b * (N // _BLKA) + i,)),
        ],
        out_shape=[
            jax.ShapeDtypeStruct((B, 1, N), jnp.float32),
            jax.ShapeDtypeStruct((B * N,), jnp.int32),
        ],
    )(scores)


_BLKF = 8192
_BLKX = 16384


def _flat_xyz_body(x_ref, ox, oy, oz):
    ox[...] = x_ref[0, 0, :]
    oy[...] = x_ref[0, 1, :]
    oz[...] = x_ref[0, 2, :]


def _flat_xyz(xyz_t):
    return pl.pallas_call(
        _flat_xyz_body,
        grid=(B, N // _BLKX),
        in_specs=[pl.BlockSpec((1, 3, _BLKX), lambda b, i: (b, 0, i))],
        out_specs=[pl.BlockSpec((_BLKX,), lambda b, i: (b * (N // _BLKX) + i,))
                   for _ in range(3)],
        out_shape=[jax.ShapeDtypeStruct((B * N,), jnp.float32)
                   for _ in range(3)],
    )(xyz_t)


def _flat_feat_body(f_ref, *out_refs):
    for c in range(C):
        out_refs[c][...] = f_ref[0, c, :]


def _flat_feat(features):
    return pl.pallas_call(
        _flat_feat_body,
        grid=(B, N // _BLKF),
        in_specs=[pl.BlockSpec((1, C, _BLKF), lambda b, i: (b, 0, i))],
        out_specs=[pl.BlockSpec((_BLKF,), lambda b, i: (b * (N // _BLKF) + i,))
                   for _ in range(C)],
        out_shape=[jax.ShapeDtypeStruct((B * N,), jnp.float32)
                   for _ in range(C)],
    )(features)


def _u32(x):
    return plsc.bitcast(x, jnp.uint32)


def _i32(x):
    return plsc.bitcast(x, jnp.int32)


def _scal(vec, lane):
    it = lax.iota(jnp.int32, 16)
    return jnp.max(jnp.where(it == lane, vec, jnp.int32(-1)))


_SC_MESH = plsc.VectorSubcoreMesh(core_axis_name="c", subcore_axis_name="s")


@functools.partial(
    pl.kernel,
    out_type=(
        jax.ShapeDtypeStruct((B, K), jnp.int32),
        jax.ShapeDtypeStruct((B, K), jnp.int32),
    ),
    mesh=_SC_MESH,
    compiler_params=pltpu.CompilerParams(needs_layout_passes=False),
    scratch_types=dict(
        keys_t=pltpu.VMEM((CH,), jnp.int32),
        hist_l=pltpu.VMEM((2048,), jnp.int32),
        hidx=pltpu.VMEM((2048,), jnp.int32),
        klt=pltpu.VMEM((CAP,), jnp.int32),
        ilt=pltpu.VMEM((CAP,), jnp.int32),
        ieq=pltpu.VMEM((CAP,), jnp.int32),
        dst_c=pltpu.VMEM((4096,), jnp.int32),
        v16=pltpu.VMEM((16,), jnp.int32),
        selk_sh=pltpu.VMEM_SHARED((4 * SEL + 16,), jnp.int32),
        seli_sh=pltpu.VMEM_SHARED((4 * SEL + 16,), jnp.int32),
        hist_sh=pltpu.VMEM_SHARED((4 * 4 * 2048,), jnp.int32),
        pongk_sh=pltpu.VMEM_SHARED((4 * SEL + 16,), jnp.int32),
        pongi_sh=pltpu.VMEM_SHARED((4 * SEL + 16,), jnp.int32),
        scal_sh=pltpu.VMEM_SHARED((4 * 16,), jnp.int32),
        cnt_sh=pltpu.VMEM_SHARED((4 * 4 * 16,), jnp.int32),
        sem=pltpu.SemaphoreType.DMA,
    ),
)
def _select_sort(keys_hbm, sidx_hbm, tsc_hbm, keys_t, hist_l, hidx, klt, ilt,
                 ieq, dst_c, v16, selk_sh, seli_sh, hist_sh, pongk_sh,
                 pongi_sh, scal_sh, cnt_sh, sem):
    cid = lax.axis_index("c")
    sid = lax.axis_index("s")
    it = lax.iota(jnp.int32, 16)
    bl = sid // NT
    b = cid * 4 + bl
    q = sid % NT
    r0 = q * CH
    is_lead = q == 0

    pltpu.sync_copy(keys_hbm.at[pl.ds(b * N + r0, CH)], keys_t)

    zero16 = jnp.zeros((16,), jnp.int32)

    def digits(x_i32):
        xu = _u32(x_i32)
        d1 = (xu >> 21).astype(jnp.int32)
        d2 = ((xu >> 10) & jnp.uint32(0x7FF)).astype(jnp.int32)
        d3 = (xu & jnp.uint32(0x3FF)).astype(jnp.int32)
        return d1, d2, d3

    def _mkhidx(j, _):
        hidx[pl.ds(j * 16, 16)] = bl * 8192 + j * 16 + it
        return 0
    lax.fori_loop(0, 128, _mkhidx, 0)

    def level_hist(lv, b1, b2):
        def body(j, _):
            for u in range(4):
                x = keys_t[pl.ds((j * 4 + u) * 16, 16)]
                d1, d2, d3 = digits(x)
                if lv == 0:
                    d, cond = d1, None
                elif lv == 1:
                    d = d2
                    cond = d1 == b1
                else:
                    d = d3
                    cond = jnp.logical_and(d1 == b1, d2 == b2)
                cnt, last = plsc.scan_count(d, mask=cond)
                plsc.addupdate_scatter(hist_l, [d], cnt, mask=last)
            return 0
        lax.fori_loop(0, CH // 64, body, 0)

    def find_bin(nbins, target):
        def body(i, carry):
            found, bd, cb, total = carry
            h = hist_l[pl.ds(i * 16, 16)]
            cs = plsc.cumsum(h)
            hit = (total + cs) >= target
            f = jnp.min(jnp.where(hit, it, jnp.int32(16)))
            fnow = (f < 16).astype(jnp.int32)
            fc = jnp.minimum(f, 15)
            e = jnp.max(jnp.where(it == fc, cs - h, jnp.int32(-(2**30))))
            upd = fnow * (1 - found)
            bd = jnp.where(upd == 1, i * 16 + f, bd)
            cb = jnp.where(upd == 1, total + e, cb)
            total = total + jnp.max(cs)
            return (jnp.maximum(found, fnow), bd, cb, total)
        res = lax.fori_loop(0, nbins // 16, body,
                            (jnp.int32(0), jnp.int32(0), jnp.int32(0),
                             jnp.int32(0)))
        return res[1], res[2]

    b1 = jnp.int32(0)
    b2 = jnp.int32(0)
    b3 = jnp.int32(0)
    cnt_lt = jnp.int32(0)
    target = jnp.int32(K)
    for lv in range(3):
        nbins = 1024 if lv == 2 else 2048
        def _zh(j, _):
            hist_l[pl.ds(j * 16, 16)] = zero16
            return 0
        lax.fori_loop(0, 128, _zh, 0)
        @pl.when(is_lead)
        def _():
            pltpu.sync_copy(hist_l, hist_sh.at[pl.ds(bl * 8192, 2048)])
        if lv > 0:
            pltpu.sync_copy(scal_sh.at[pl.ds(bl * 16, 16)], v16)
            sv = v16[...]
            b1 = _scal(sv, 0)
            b2 = _scal(sv, 1)
            target = _scal(sv, 3)
        level_hist(lv, b1, b2)
        plsc.subcore_barrier()
        pltpu.sync_copy(hist_l, hist_sh.at[hidx], add=True)
        plsc.subcore_barrier()
        @pl.when(is_lead)
        def _():
            pltpu.sync_copy(hist_sh.at[pl.ds(bl * 8192, 2048)], hist_l)
        bd, cb = find_bin(nbins, target)
        if lv == 0:
            b1n, b2n, b3n = bd, jnp.int32(0), jnp.int32(0)
        elif lv == 1:
            b1n, b2n, b3n = b1, bd, jnp.int32(0)
        else:
            b1n, b2n, b3n = b1, b2, bd
        cnt_lt = cnt_lt + cb if lv > 0 else cb
        target_n = target - cb
        @pl.when(is_lead)
        def _():
            out = jnp.where(it == 0, b1n, zero16)
            out = jnp.where(it == 1, b2n, out)
            out = jnp.where(it == 2, b3n, out)
            out = jnp.where(it == 3, target_n, out)
            out = jnp.where(it == 4, (cnt_lt if lv == 2 else jnp.int32(0)),
                            out)
            v16[...] = out
            pltpu.sync_copy(v16, scal_sh.at[pl.ds(bl * 16, 16)])
        plsc.subcore_barrier()

    pltpu.sync_copy(scal_sh.at[pl.ds(bl * 16, 16)], v16)
    sv = v16[...]
    b1 = _scal(sv, 0)
    b2 = _scal(sv, 1)
    b3 = _scal(sv, 2)
    lt_tot = _scal(sv, 4)
    needed_eq = K - lt_tot

    def comp_body(j, carry):
        off_lt, off_eq = carry
        for u in range(4):
            x = keys_t[pl.ds((j * 4 + u) * 16, 16)]
            d1, d2, d3 = digits(x)
            e1 = d1 == b1
            e2 = jnp.logical_and(e1, d2 == b2)
            lt = jnp.logical_or(
                d1 < b1,
                jnp.logical_or(jnp.logical_and(e1, d2 < b2),
                               jnp.logical_and(e2, d3 < b3)))
            eq = jnp.logical_and(e2, d3 == b3)
            idxv = r0 + (j * 4 + u) * 16 + it
            plsc.store_compressed(klt.at[pl.ds(off_lt, 16)], x, mask=lt)
            plsc.store_compressed(ilt.at[pl.ds(off_lt, 16)], idxv, mask=lt)
            pos = off_eq + plsc.cumsum(eq.astype(jnp.int32)) - 1
            eq2 = jnp.logical_and(eq, pos < SEL)
            off_eq_c = jnp.minimum(off_eq, jnp.int32(SEL - 16))
            plsc.store_compressed(ieq.at[pl.ds(off_eq_c, 16)], idxv, mask=eq2)
            off_lt = off_lt + jnp.sum(lt.astype(jnp.int32))
            off_eq = off_eq + jnp.sum(eq2.astype(jnp.int32))
        return (off_lt, off_eq)
    off_lt, off_eq = lax.fori_loop(0, CH // 64, comp_body,
                                   (jnp.int32(0), jnp.int32(0)))

    out = jnp.where(it == 0, off_lt, zero16)
    out = jnp.where(it == 1, off_eq, out)
    v16[...] = out
    pltpu.sync_copy(v16, cnt_sh.at[pl.ds((bl * 4 + q) * 16, 16)])
    plsc.subcore_barrier()

    lt_pref = jnp.int32(0)
    eq_pref = jnp.int32(0)
    for t in range(NT):
        pltpu.sync_copy(cnt_sh.at[pl.ds((bl * 4 + t) * 16, 16)], v16)
        vv = v16[...]
        lt_t = _scal(vv, 0)
        eq_t = _scal(vv, 1)
        tq = jnp.int32(t) < q
        lt_pref = lt_pref + jnp.where(tq, lt_t, 0)
        eq_pref = eq_pref + jnp.where(tq, eq_t, 0)

    sel_base = bl * SEL
    for ch in range(4):
        cb0 = ch * 4096
        @pl.when(off_lt > cb0)
        def _():
            def mkd(j, _):
                p = cb0 + j * 16 + it
                dst_c[pl.ds(j * 16, 16)] = jnp.where(
                    p < off_lt, sel_base + lt_pref + p, jnp.int32(DUMP))
                return 0
            lax.fori_loop(0, 256, mkd, 0)
            pltpu.sync_copy(klt.at[pl.ds(cb0, 4096)], selk_sh.at[dst_c])
            pltpu.sync_copy(ilt.at[pl.ds(cb0, 4096)], seli_sh.at[dst_c])
        @pl.when(jnp.logical_and(off_eq > cb0, needed_eq > eq_pref + cb0))
        def _():
            def mkd(j, _):
                p = cb0 + j * 16 + it
                g = eq_pref + p
                ok = jnp.logical_and(p < off_eq, g < needed_eq)
                dst_c[pl.ds(j * 16, 16)] = jnp.where(
                    ok, sel_base + lt_tot + g, jnp.int32(DUMP))
                return 0
            lax.fori_loop(0, 256, mkd, 0)
            pltpu.sync_copy(ieq.at[pl.ds(cb0, 4096)], seli_sh.at[dst_c])
    plsc.subcore_barrier()

    t_raw = jnp.left_shift(b1, 21) | jnp.left_shift(b2, 10) | b3
    qb = q * (K // NT)

    pings = (selk_sh, seli_sh, pongk_sh, pongi_sh)
    for pno, (shift, dmask, nbins) in enumerate(
            ((0, 0x3FF, 1024), (10, 0x7FF, 2048), (21, 0x7FF, 2048))):
        src_k_sh, src_i_sh = pings[(pno % 2) * 2:(pno % 2) * 2 + 2]
        dst_k_sh, dst_i_sh = pings[((pno + 1) % 2) * 2:((pno + 1) % 2) * 2 + 2]
        pltpu.sync_copy(src_k_sh.at[pl.ds(sel_base + qb, K // NT)],
                        klt.at[pl.ds(0, K // NT)])
        pltpu.sync_copy(src_i_sh.at[pl.ds(sel_base + qb, K // NT)],
                        ilt.at[pl.ds(0, K // NT)])
        if pno == 0:
            def fill_tail(j, _):
                for u in range(4):
                    p = qb + (j * 4 + u) * 16 + it
                    x = klt[pl.ds((j * 4 + u) * 16, 16)]
                    klt[pl.ds((j * 4 + u) * 16, 16)] = jnp.where(
                        p >= lt_tot, t_raw, x)
                return 0
            lax.fori_loop(0, K // NT // 64, fill_tail, 0)

        def _zh(j, _):
            hist_l[pl.ds(j * 16, 16)] = zero16
            return 0
        lax.fori_loop(0, nbins // 16, _zh, 0)

        def hist_body(j, _):
            for u in range(4):
                x = klt[pl.ds((j * 4 + u) * 16, 16)]
                d = ((_u32(x) >> shift) & jnp.uint32(dmask)).astype(jnp.int32)
                cnt, last = plsc.scan_count(d)
                plsc.addupdate_scatter(hist_l, [d], cnt, mask=last)
            return 0
        lax.fori_loop(0, K // NT // 64, hist_body, 0)

        pltpu.sync_copy(hist_l.at[pl.ds(0, nbins)],
                        hist_sh.at[pl.ds(bl * 8192 + q * 2048, nbins)])
        plsc.subcore_barrier()

        def off_body(j, carry):
            hs = []
            for t in range(NT):
                pltpu.sync_copy(
                    hist_sh.at[pl.ds(bl * 8192 + t * 2048 + j * 16, 16)], v16)
                hs.append(v16[...])
            tot = hs[0] + hs[1] + hs[2] + hs[3]
            cs = plsc.cumsum(tot)
            excl = carry + cs - tot
            own = excl
            for t in range(NT):
                own = own + jnp.where(jnp.int32(t) < q, hs[t], 0)
            hist_l[pl.ds(j * 16, 16)] = own
            return carry + jnp.max(cs)
        lax.fori_loop(0, nbins // 16, off_body, jnp.int32(0))

        def scat_body(j, _):
            for u in range(2):
                x = klt[pl.ds((j * 2 + u) * 16, 16)]
                v = ilt[pl.ds((j * 2 + u) * 16, 16)]
                d = ((_u32(x) >> shift) & jnp.uint32(dmask)).astype(jnp.int32)
                cnt, last = plsc.scan_count(d)
                base = plsc.load_gather(hist_l, [d])
                dst = base + cnt - 1
                plsc.store_scatter(dst_c, [(j * 2 + u) * 16 + it],
                                   sel_base + dst)
                plsc.addupdate_scatter(hist_l, [d], cnt, mask=last)
            return 0
        lax.fori_loop(0, K // NT // 32, scat_body, 0)
        pltpu.sync_copy(klt.at[pl.ds(0, K // NT)], dst_k_sh.at[dst_c])
        pltpu.sync_copy(ilt.at[pl.ds(0, K // NT)], dst_i_sh.at[dst_c])
        plsc.subcore_barrier()

    pltpu.sync_copy(pongi_sh.at[pl.ds(sel_base + qb, K // NT)],
                    ilt.at[pl.ds(0, K // NT)])
    pltpu.sync_copy(ilt.at[pl.ds(0, K // NT)],
                    sidx_hbm.at[b, pl.ds(qb, K // NT)])
    pltpu.sync_copy(pongk_sh.at[pl.ds(sel_base + qb, K // NT)],
                    klt.at[pl.ds(0, K // NT)])

    def score_body(j, _):
        for u in range(4):
            kx = _u32(klt[pl.ds((j * 4 + u) * 16, 16)])
            neg = (kx >> 31) == 1
            bu = jnp.where(neg, kx, ~kx & jnp.uint32(0x7FFFFFFF))
            klt[pl.ds((j * 4 + u) * 16, 16)] = _i32(bu)
        return 0
    lax.fori_loop(0, K // NT // 64, score_body, 0)
    pltpu.sync_copy(klt.at[pl.ds(0, K // NT)],
                    tsc_hbm.at[b, pl.ds(qb, K // NT)])


@functools.partial(
    pl.kernel,
    out_type=tuple(
        [jax.ShapeDtypeStruct((B, K), jnp.float32) for _ in range(3)]
        + [jax.ShapeDtypeStruct((B, C * K), jnp.float32)]),
    mesh=_SC_MESH,
    compiler_params=pltpu.CompilerParams(needs_layout_passes=False),
    scratch_types=dict(
        idxb=pltpu.VMEM((QCH,), jnp.int32),
        fidx=pltpu.VMEM((QCH,), jnp.int32),
        fbuf=pltpu.VMEM((QCH,), jnp.float32),
        fbuf2=pltpu.VMEM((QCH,), jnp.float32),
        sem=pltpu.SemaphoreType.DMA,
        sem2=pltpu.SemaphoreType.DMA,
    ),
)
def _gather(sidx_hbm, *rest, idxb, fidx, fbuf, fbuf2, sem, sem2):
    xyz_refs = rest[:3]
    feat_refs = rest[3:3 + C]
    px_hbm, py_hbm, pz_hbm, nf_hbm = rest[3 + C:]
    cid = lax.axis_index("c")
    sid = lax.axis_index("s")
    b = cid * 4 + sid // NT
    q = sid % NT
    r0 = q * QCH

    pltpu.sync_copy(sidx_hbm.at[b, pl.ds(r0, QCH)], idxb)

    def mkf(j, _):
        fidx[pl.ds(j * 16, 16)] = b * N + idxb[pl.ds(j * 16, 16)]
        return 0
    lax.fori_loop(0, QCH // 16, mkf, 0)

    srcs = list(xyz_refs) + list(feat_refs)
    dsts = ([(o, b, r0) for o in (px_hbm, py_hbm, pz_hbm)]
            + [(nf_hbm, b, c * K + r0) for c in range(C)])
    bufs = (fbuf, fbuf2)
    hs = [None, None]
    hs[0] = pltpu.async_copy(srcs[0].at[fidx], bufs[0], sem)
    for i in range(len(srcs)):
        if i + 1 < len(srcs):
            hs[(i + 1) % 2] = pltpu.async_copy(
                srcs[i + 1].at[fidx], bufs[(i + 1) % 2], sem2)
        hs[i % 2].wait()
        o_hbm, bb, off = dsts[i]
        pltpu.sync_copy(bufs[i % 2], o_hbm.at[bb, pl.ds(off, QCH)])


def kernel(xyz, features, scores):
    ori, keys = _prep(scores)
    xyz_flat = _flat_xyz(jnp.transpose(xyz, (0, 2, 1)))
    feat_flat = _flat_feat(features)
    sidx, tsc = _select_sort(keys)
    px, py, pz, nf = _gather(sidx, *xyz_flat, *feat_flat)
    new_xyz = jnp.stack((px, py, pz), axis=2)
    new_features = nf.reshape(B, C, K)
    top_scores = lax.bitcast_convert_type(tsc, jnp.float32)
    return new_xyz, new_features, top_scores, ori


# speedup vs baseline: 1.2424x; 1.2424x over previous
import functools
import jax
import jax.numpy as jnp
from jax import lax
from jax.experimental import pallas as pl
from jax.experimental.pallas import tpu as pltpu
from jax.experimental.pallas import tpu_sc as plsc

B = 8
N = 131072
C = 16
K = 16384
NT = 4
CH = N // NT
SEL = K
CAP = SEL + 16
DUMP = 4 * SEL
QCH = K // NT

_BLKA = 16384


def _prep_body(scores_ref, ori_ref, keys_ref):
    s = scores_ref[...]
    m = jnp.max(s, axis=1)
    ori_ref[...] = jax.nn.sigmoid(m)[:, None, :]
    bu = lax.bitcast_convert_type(m, jnp.uint32)
    sign = bu >> 31
    key = jnp.where(sign == 1, bu, ~bu & jnp.uint32(0x7FFFFFFF))
    keys_ref[...] = lax.bitcast_convert_type(key, jnp.int32).reshape(_BLKA)


def _prep(scores):
    return pl.pallas_call(
        _prep_body,
        grid=(B, N // _BLKA),
        in_specs=[pl.BlockSpec((1, 3, _BLKA), lambda b, i: (b, 0, i))],
        out_specs=[
            pl.BlockSpec((1, 1, _BLKA), lambda b, i: (b, 0, i)),
            pl.BlockSpec((_BLKA,), lambda b, i: (b * (N // _BLKA) + i,)),
        ],
        out_shape=[
            jax.ShapeDtypeStruct((B, 1, N), jnp.float32),
            jax.ShapeDtypeStruct((B * N,), jnp.int32),
        ],
    )(scores)


_BLKF = 8192
_BLKX = 16384


def _flat_xyz_body(x_ref, ox, oy, oz):
    ox[...] = x_ref[0, 0, :]
    oy[...] = x_ref[0, 1, :]
    oz[...] = x_ref[0, 2, :]


def _flat_xyz(xyz_t):
    return pl.pallas_call(
        _flat_xyz_body,
        grid=(B, N // _BLKX),
        in_specs=[pl.BlockSpec((1, 3, _BLKX), lambda b, i: (b, 0, i))],
        out_specs=[pl.BlockSpec((_BLKX,), lambda b, i: (b * (N // _BLKX) + i,))
                   for _ in range(3)],
        out_shape=[jax.ShapeDtypeStruct((B * N,), jnp.float32)
                   for _ in range(3)],
    )(xyz_t)


def _flat_feat_body(f_ref, *out_refs):
    for c in range(C):
        out_refs[c][...] = f_ref[0, c, :]


def _flat_feat(features):
    return pl.pallas_call(
        _flat_feat_body,
        grid=(B, N // _BLKF),
        in_specs=[pl.BlockSpec((1, C, _BLKF), lambda b, i: (b, 0, i))],
        out_specs=[pl.BlockSpec((_BLKF,), lambda b, i: (b * (N // _BLKF) + i,))
                   for _ in range(C)],
        out_shape=[jax.ShapeDtypeStruct((B * N,), jnp.float32)
                   for _ in range(C)],
    )(features)


def _u32(x):
    return plsc.bitcast(x, jnp.uint32)


def _i32(x):
    return plsc.bitcast(x, jnp.int32)


def _scal(vec, lane):
    it = lax.iota(jnp.int32, 16)
    return jnp.max(jnp.where(it == lane, vec, jnp.int32(-1)))


_SC_MESH = plsc.VectorSubcoreMesh(core_axis_name="c", subcore_axis_name="s")


@functools.partial(
    pl.kernel,
    out_type=(
        jax.ShapeDtypeStruct((B, K), jnp.int32),
        jax.ShapeDtypeStruct((B, K), jnp.int32),
    ),
    mesh=_SC_MESH,
    compiler_params=pltpu.CompilerParams(needs_layout_passes=False),
    scratch_types=dict(
        keys_t=pltpu.VMEM((CH,), jnp.int32),
        hist_l=pltpu.VMEM((2048,), jnp.int32),
        hidx=pltpu.VMEM((2048,), jnp.int32),
        klt=pltpu.VMEM((CAP,), jnp.int32),
        ilt=pltpu.VMEM((CAP,), jnp.int32),
        ieq=pltpu.VMEM((CAP,), jnp.int32),
        dst_c=pltpu.VMEM((4096,), jnp.int32),
        v16=pltpu.VMEM((16,), jnp.int32),
        selk_sh=pltpu.VMEM_SHARED((4 * SEL + 16,), jnp.int32),
        seli_sh=pltpu.VMEM_SHARED((4 * SEL + 16,), jnp.int32),
        hist_sh=pltpu.VMEM_SHARED((4 * 4 * 2048,), jnp.int32),
        pongk_sh=pltpu.VMEM_SHARED((4 * SEL + 16,), jnp.int32),
        pongi_sh=pltpu.VMEM_SHARED((4 * SEL + 16,), jnp.int32),
        scal_sh=pltpu.VMEM_SHARED((4 * 16,), jnp.int32),
        cnt_sh=pltpu.VMEM_SHARED((4 * 4 * 16,), jnp.int32),
        sem=pltpu.SemaphoreType.DMA,
    ),
)
def _select_sort(keys_hbm, sidx_hbm, tsc_hbm, keys_t, hist_l, hidx, klt, ilt,
                 ieq, dst_c, v16, selk_sh, seli_sh, hist_sh, pongk_sh,
                 pongi_sh, scal_sh, cnt_sh, sem):
    cid = lax.axis_index("c")
    sid = lax.axis_index("s")
    it = lax.iota(jnp.int32, 16)
    bl = sid // NT
    b = cid * 4 + bl
    q = sid % NT
    r0 = q * CH
    is_lead = q == 0

    pltpu.sync_copy(keys_hbm.at[pl.ds(b * N + r0, CH)], keys_t)

    zero16 = jnp.zeros((16,), jnp.int32)

    def digits(x_i32):
        xu = _u32(x_i32)
        d1 = (xu >> 21).astype(jnp.int32)
        d2 = ((xu >> 10) & jnp.uint32(0x7FF)).astype(jnp.int32)
        d3 = (xu & jnp.uint32(0x3FF)).astype(jnp.int32)
        return d1, d2, d3

    def _mkhidx(j, _):
        hidx[pl.ds(j * 16, 16)] = bl * 8192 + j * 16 + it
        return 0
    lax.fori_loop(0, 128, _mkhidx, 0)

    def level_hist(lv, b1, b2):
        def body(j, _):
            for u in range(4):
                x = keys_t[pl.ds((j * 4 + u) * 16, 16)]
                d1, d2, d3 = digits(x)
                if lv == 0:
                    d, cond = d1, None
                elif lv == 1:
                    d = d2
                    cond = d1 == b1
                else:
                    d = d3
                    cond = jnp.logical_and(d1 == b1, d2 == b2)
                cnt, last = plsc.scan_count(d, mask=cond)
                plsc.addupdate_scatter(hist_l, [d], cnt, mask=last)
            return 0
        lax.fori_loop(0, CH // 64, body, 0)

    def find_bin(nbins, target):
        def body(i, carry):
            found, bd, cb, total = carry
            h = hist_l[pl.ds(i * 16, 16)]
            cs = plsc.cumsum(h)
            hit = (total + cs) >= target
            f = jnp.min(jnp.where(hit, it, jnp.int32(16)))
            fnow = (f < 16).astype(jnp.int32)
            fc = jnp.minimum(f, 15)
            e = jnp.max(jnp.where(it == fc, cs - h, jnp.int32(-(2**30))))
            upd = fnow * (1 - found)
            bd = jnp.where(upd == 1, i * 16 + f, bd)
            cb = jnp.where(upd == 1, total + e, cb)
            total = total + jnp.max(cs)
            return (jnp.maximum(found, fnow), bd, cb, total)
        res = lax.fori_loop(0, nbins // 16, body,
                            (jnp.int32(0), jnp.int32(0), jnp.int32(0),
                             jnp.int32(0)))
        return res[1], res[2]

    b1 = jnp.int32(0)
    b2 = jnp.int32(0)
    b3 = jnp.int32(0)
    cnt_lt = jnp.int32(0)
    target = jnp.int32(K)
    for lv in range(3):
        nbins = 1024 if lv == 2 else 2048
        def _zh(j, _):
            hist_l[pl.ds(j * 16, 16)] = zero16
            return 0
        lax.fori_loop(0, 128, _zh, 0)
        @pl.when(is_lead)
        def _():
            pltpu.sync_copy(hist_l, hist_sh.at[pl.ds(bl * 8192, 2048)])
        if lv > 0:
            pltpu.sync_copy(scal_sh.at[pl.ds(bl * 16, 16)], v16)
            sv = v16[...]
            b1 = _scal(sv, 0)
            b2 = _scal(sv, 1)
            target = _scal(sv, 3)
        level_hist(lv, b1, b2)
        plsc.subcore_barrier()
        pltpu.sync_copy(hist_l, hist_sh.at[hidx], add=True)
        plsc.subcore_barrier()
        @pl.when(is_lead)
        def _():
            pltpu.sync_copy(hist_sh.at[pl.ds(bl * 8192, 2048)], hist_l)
        bd, cb = find_bin(nbins, target)
        if lv == 0:
            b1n, b2n, b3n = bd, jnp.int32(0), jnp.int32(0)
        elif lv == 1:
            b1n, b2n, b3n = b1, bd, jnp.int32(0)
        else:
            b1n, b2n, b3n = b1, b2, bd
        cnt_lt = cnt_lt + cb if lv > 0 else cb
        target_n = target - cb
        @pl.when(is_lead)
        def _():
            out = jnp.where(it == 0, b1n, zero16)
            out = jnp.where(it == 1, b2n, out)
            out = jnp.where(it == 2, b3n, out)
            out = jnp.where(it == 3, target_n, out)
            out = jnp.where(it == 4, (cnt_lt if lv == 2 else jnp.int32(0)),
                            out)
            v16[...] = out
            pltpu.sync_copy(v16, scal_sh.at[pl.ds(bl * 16, 16)])
        plsc.subcore_barrier()

    pltpu.sync_copy(scal_sh.at[pl.ds(bl * 16, 16)], v16)
    sv = v16[...]
    b1 = _scal(sv, 0)
    b2 = _scal(sv, 1)
    b3 = _scal(sv, 2)
    lt_tot = _scal(sv, 4)
    needed_eq = K - lt_tot

    def comp_body(j, carry):
        off_lt, off_eq = carry
        for u in range(4):
            x = keys_t[pl.ds((j * 4 + u) * 16, 16)]
            d1, d2, d3 = digits(x)
            e1 = d1 == b1
            e2 = jnp.logical_and(e1, d2 == b2)
            lt = jnp.logical_or(
                d1 < b1,
                jnp.logical_or(jnp.logical_and(e1, d2 < b2),
                               jnp.logical_and(e2, d3 < b3)))
            eq = jnp.logical_and(e2, d3 == b3)
            idxv = r0 + (j * 4 + u) * 16 + it
            plsc.store_compressed(klt.at[pl.ds(off_lt, 16)], x, mask=lt)
            plsc.store_compressed(ilt.at[pl.ds(off_lt, 16)], idxv, mask=lt)
            pos = off_eq + plsc.cumsum(eq.astype(jnp.int32)) - 1
            eq2 = jnp.logical_and(eq, pos < SEL)
            off_eq_c = jnp.minimum(off_eq, jnp.int32(SEL - 16))
            plsc.store_compressed(ieq.at[pl.ds(off_eq_c, 16)], idxv, mask=eq2)
            off_lt = off_lt + jnp.sum(lt.astype(jnp.int32))
            off_eq = off_eq + jnp.sum(eq2.astype(jnp.int32))
        return (off_lt, off_eq)
    off_lt, off_eq = lax.fori_loop(0, CH // 64, comp_body,
                                   (jnp.int32(0), jnp.int32(0)))

    out = jnp.where(it == 0, off_lt, zero16)
    out = jnp.where(it == 1, off_eq, out)
    v16[...] = out
    pltpu.sync_copy(v16, cnt_sh.at[pl.ds((bl * 4 + q) * 16, 16)])
    plsc.subcore_barrier()

    lt_pref = jnp.int32(0)
    eq_pref = jnp.int32(0)
    for t in range(NT):
        pltpu.sync_copy(cnt_sh.at[pl.ds((bl * 4 + t) * 16, 16)], v16)
        vv = v16[...]
        lt_t = _scal(vv, 0)
        eq_t = _scal(vv, 1)
        tq = jnp.int32(t) < q
        lt_pref = lt_pref + jnp.where(tq, lt_t, 0)
        eq_pref = eq_pref + jnp.where(tq, eq_t, 0)

    sel_base = bl * SEL
    for ch in range(4):
        cb0 = ch * 4096
        @pl.when(off_lt > cb0)
        def _():
            def mkd(j, _):
                p = cb0 + j * 16 + it
                dst_c[pl.ds(j * 16, 16)] = jnp.where(
                    p < off_lt, sel_base + lt_pref + p, jnp.int32(DUMP))
                return 0
            lax.fori_loop(0, 256, mkd, 0)
            pltpu.sync_copy(klt.at[pl.ds(cb0, 4096)], selk_sh.at[dst_c])
            pltpu.sync_copy(ilt.at[pl.ds(cb0, 4096)], seli_sh.at[dst_c])
        @pl.when(jnp.logical_and(off_eq > cb0, needed_eq > eq_pref + cb0))
        def _():
            def mkd(j, _):
                p = cb0 + j * 16 + it
                g = eq_pref + p
                ok = jnp.logical_and(p < off_eq, g < needed_eq)
                dst_c[pl.ds(j * 16, 16)] = jnp.where(
                    ok, sel_base + lt_tot + g, jnp.int32(DUMP))
                return 0
            lax.fori_loop(0, 256, mkd, 0)
            pltpu.sync_copy(ieq.at[pl.ds(cb0, 4096)], seli_sh.at[dst_c])
    plsc.subcore_barrier()

    t_raw = jnp.left_shift(b1, 21) | jnp.left_shift(b2, 10) | b3
    qb = q * (K // NT)

    pings = (selk_sh, seli_sh, pongk_sh, pongi_sh)
    for pno, (shift, dmask, nbins) in enumerate(
            ((0, 0x3FF, 1024), (10, 0x7FF, 2048), (21, 0x7FF, 2048))):
        src_k_sh, src_i_sh = pings[(pno % 2) * 2:(pno % 2) * 2 + 2]
        dst_k_sh, dst_i_sh = pings[((pno + 1) % 2) * 2:((pno + 1) % 2) * 2 + 2]
        pltpu.sync_copy(src_k_sh.at[pl.ds(sel_base + qb, K // NT)],
                        klt.at[pl.ds(0, K // NT)])
        pltpu.sync_copy(src_i_sh.at[pl.ds(sel_base + qb, K // NT)],
                        ilt.at[pl.ds(0, K // NT)])
        if pno == 0:
            def fill_tail(j, _):
                for u in range(4):
                    p = qb + (j * 4 + u) * 16 + it
                    x = klt[pl.ds((j * 4 + u) * 16, 16)]
                    klt[pl.ds((j * 4 + u) * 16, 16)] = jnp.where(
                        p >= lt_tot, t_raw, x)
                return 0
            lax.fori_loop(0, K // NT // 64, fill_tail, 0)

        def _zh(j, _):
            hist_l[pl.ds(j * 16, 16)] = zero16
            return 0
        lax.fori_loop(0, nbins // 16, _zh, 0)

        def hist_body(j, _):
            for u in range(4):
                x = klt[pl.ds((j * 4 + u) * 16, 16)]
                d = ((_u32(x) >> shift) & jnp.uint32(dmask)).astype(jnp.int32)
                cnt, last = plsc.scan_count(d)
                plsc.addupdate_scatter(hist_l, [d], cnt, mask=last)
            return 0
        lax.fori_loop(0, K // NT // 64, hist_body, 0)

        pltpu.sync_copy(hist_l.at[pl.ds(0, nbins)],
                        hist_sh.at[pl.ds(bl * 8192 + q * 2048, nbins)])
        plsc.subcore_barrier()

        for t in range(NT):
            pltpu.sync_copy(
                hist_sh.at[pl.ds(bl * 8192 + t * 2048, nbins)],
                klt.at[pl.ds(4096 + t * 2048, nbins)])

        def off_body(j, carry):
            hs = [klt[pl.ds(4096 + t * 2048 + j * 16, 16)]
                  for t in range(NT)]
            tot = hs[0] + hs[1] + hs[2] + hs[3]
            cs = plsc.cumsum(tot)
            own = carry + cs - tot
            for t in range(NT):
                own = own + jnp.where(jnp.int32(t) < q, hs[t], 0)
            hist_l[pl.ds(j * 16, 16)] = own
            return carry + jnp.max(cs)
        lax.fori_loop(0, nbins // 16, off_body, jnp.int32(0))

        def scat_body(j, _):
            for u in range(2):
                x = klt[pl.ds((j * 2 + u) * 16, 16)]
                v = ilt[pl.ds((j * 2 + u) * 16, 16)]
                d = ((_u32(x) >> shift) & jnp.uint32(dmask)).astype(jnp.int32)
                cnt, last = plsc.scan_count(d)
                base = plsc.load_gather(hist_l, [d])
                dst = base + cnt - 1
                plsc.store_scatter(dst_c, [(j * 2 + u) * 16 + it],
                                   sel_base + dst)
                plsc.addupdate_scatter(hist_l, [d], cnt, mask=last)
            return 0
        lax.fori_loop(0, K // NT // 32, scat_body, 0)
        pltpu.sync_copy(klt.at[pl.ds(0, K // NT)], dst_k_sh.at[dst_c])
        pltpu.sync_copy(ilt.at[pl.ds(0, K // NT)], dst_i_sh.at[dst_c])
        plsc.subcore_barrier()

    pltpu.sync_copy(pongi_sh.at[pl.ds(sel_base + qb, K // NT)],
                    ilt.at[pl.ds(0, K // NT)])
    pltpu.sync_copy(ilt.at[pl.ds(0, K // NT)],
                    sidx_hbm.at[b, pl.ds(qb, K // NT)])
    pltpu.sync_copy(pongk_sh.at[pl.ds(sel_base + qb, K // NT)],
                    klt.at[pl.ds(0, K // NT)])

    def score_body(j, _):
        for u in range(4):
            kx = _u32(klt[pl.ds((j * 4 + u) * 16, 16)])
            neg = (kx >> 31) == 1
            bu = jnp.where(neg, kx, ~kx & jnp.uint32(0x7FFFFFFF))
            klt[pl.ds((j * 4 + u) * 16, 16)] = _i32(bu)
        return 0
    lax.fori_loop(0, K // NT // 64, score_body, 0)
    pltpu.sync_copy(klt.at[pl.ds(0, K // NT)],
                    tsc_hbm.at[b, pl.ds(qb, K // NT)])


@functools.partial(
    pl.kernel,
    out_type=tuple(
        [jax.ShapeDtypeStruct((B, K), jnp.float32) for _ in range(3)]
        + [jax.ShapeDtypeStruct((B, C * K), jnp.float32)]),
    mesh=_SC_MESH,
    compiler_params=pltpu.CompilerParams(needs_layout_passes=False),
    scratch_types=dict(
        idxb=pltpu.VMEM((QCH,), jnp.int32),
        fidx=pltpu.VMEM((QCH,), jnp.int32),
        fbuf=pltpu.VMEM((QCH,), jnp.float32),
        fbuf2=pltpu.VMEM((QCH,), jnp.float32),
        sem=pltpu.SemaphoreType.DMA,
        sem2=pltpu.SemaphoreType.DMA,
    ),
)
def _gather(sidx_hbm, *rest, idxb, fidx, fbuf, fbuf2, sem, sem2):
    xyz_refs = rest[:3]
    feat_refs = rest[3:3 + C]
    px_hbm, py_hbm, pz_hbm, nf_hbm = rest[3 + C:]
    cid = lax.axis_index("c")
    sid = lax.axis_index("s")
    b = cid * 4 + sid // NT
    q = sid % NT
    r0 = q * QCH

    pltpu.sync_copy(sidx_hbm.at[b, pl.ds(r0, QCH)], idxb)

    def mkf(j, _):
        fidx[pl.ds(j * 16, 16)] = b * N + idxb[pl.ds(j * 16, 16)]
        return 0
    lax.fori_loop(0, QCH // 16, mkf, 0)

    srcs = list(xyz_refs) + list(feat_refs)
    dsts = ([(o, b, r0) for o in (px_hbm, py_hbm, pz_hbm)]
            + [(nf_hbm, b, c * K + r0) for c in range(C)])
    bufs = (fbuf, fbuf2)
    hs = [None, None]
    hs[0] = pltpu.async_copy(srcs[0].at[fidx], bufs[0], sem)
    for i in range(len(srcs)):
        if i + 1 < len(srcs):
            hs[(i + 1) % 2] = pltpu.async_copy(
                srcs[i + 1].at[fidx], bufs[(i + 1) % 2], sem2)
        hs[i % 2].wait()
        o_hbm, bb, off = dsts[i]
        pltpu.sync_copy(bufs[i % 2], o_hbm.at[bb, pl.ds(off, QCH)])


def kernel(xyz, features, scores):
    ori, keys = _prep(scores)
    xyz_flat = _flat_xyz(jnp.transpose(xyz, (0, 2, 1)))
    feat_flat = _flat_feat(features)
    sidx, tsc = _select_sort(keys)
    px, py, pz, nf = _gather(sidx, *xyz_flat, *feat_flat)
    new_xyz = jnp.stack((px, py, pz), axis=2)
    new_features = nf.reshape(B, C, K)
    top_scores = lax.bitcast_convert_type(tsc, jnp.float32)
    return new_xyz, new_features, top_scores, ori
